# Initial kernel scaffold; baseline (speedup 1.0000x reference)
import jax
import jax.numpy as jnp
from jax.experimental import pallas as pl


def kernel(x, coordinates, edge_features, edge_index, rotatable_idx, batch_size, edge_w1, edge_b1, edge_w2, edge_b2, node_w1, node_b1, node_w2, node_b2, coord_w1, coord_b1, coord_w2, mlp_w1, mlp_b1, mlp_w2, mlp_b2):
    raise NotImplementedError("write your pallas kernel here")


# speedup vs baseline: 1.6068x; 1.6068x over previous
import functools

import jax
import jax.numpy as jnp
from jax import lax
from jax.experimental import pallas as pl
from jax.experimental.pallas import tpu as pltpu
from jax.experimental.pallas import tpu_sc as plsc

N = 10000
E = 320000
ND = 128
EDF = 16
HID = 128
W = 144
NLAYER = 7

NC, NS = 2, 16
NW = NC * NS
EPW = E // NW
CH = 80
NCHUNK = EPW // CH
NPC = N // NS

EB = 512
NB = 1000
RP = 1024
RPW = RP // NW

@functools.cache
def _mesh():
    return plsc.VectorSubcoreMesh(core_axis_name="c", subcore_axis_name="s",
                                  num_cores=NC, num_subcores=NS)
_f32 = jnp.float32


def _silu(v):
    return v * jax.nn.sigmoid(v)


def _bdot(a, b):
    return jnp.dot(a.astype(jnp.bfloat16), b.astype(jnp.bfloat16),
                   preferred_element_type=_f32)


def _btrunc(v):
    return v.astype(jnp.bfloat16).astype(_f32)


def _mdot(a, b):
    return jnp.dot(a, b.astype(jnp.bfloat16).astype(_f32),
                   preferred_element_type=_f32,
                   precision=jax.lax.Precision.HIGHEST)


def _hdot(a, b):
    return jnp.dot(a.astype(jnp.bfloat16).astype(_f32), b,
                   preferred_element_type=_f32,
                   precision=jax.lax.Precision.HIGHEST)


@functools.cache
def _build_sc_gather():
    @functools.partial(
        pl.kernel,
        out_type=(jax.ShapeDtypeStruct((E, W), _f32),
                  jax.ShapeDtypeStruct((E, W), _f32)),
        mesh=_mesh(),
        scratch_types=(pltpu.VMEM((NCHUNK, CH), jnp.int32),
                       pltpu.VMEM((NCHUNK, CH), jnp.int32),
                       pltpu.VMEM((CH, W), _f32),
                       pltpu.VMEM((CH, W), _f32),
                       pltpu.SemaphoreType.DMA,
                       pltpu.SemaphoreType.DMA),
        compiler_params=pltpu.CompilerParams(use_tc_tiling_on_sc=False),
    )
    def k(a_hbm, b_hbm, idx_hbm, ga_hbm, gb_hbm, idxs, idxd, bufa, bufb,
          sa, sb):
        wid = lax.axis_index("s") * NC + lax.axis_index("c")
        pltpu.sync_copy(idx_hbm.at[0, wid], idxs)
        pltpu.sync_copy(idx_hbm.at[1, wid], idxd)

        def body(j, carry):
            base = wid * EPW + j * CH
            ca = pltpu.async_copy(a_hbm.at[idxs.at[j]], bufa, sa)
            cb = pltpu.async_copy(b_hbm.at[idxd.at[j]], bufb, sb)
            ca.wait()
            cb.wait()
            pltpu.sync_copy(bufa, ga_hbm.at[pl.ds(base, CH)])
            pltpu.sync_copy(bufb, gb_hbm.at[pl.ds(base, CH)])
            return carry

        lax.fori_loop(0, NCHUNK, body, 0)

    return k


def _sc_gather(a, b, idx2):
    return _build_sc_gather()(a, b, idx2)


@functools.cache
def _build_sc_scatter():
    @functools.partial(
        pl.kernel,
        out_type=jax.ShapeDtypeStruct((NC, N, W), _f32),
        mesh=_mesh(),
        scratch_types=(pltpu.VMEM((NCHUNK, CH), jnp.int32),
                       pltpu.VMEM((CH, W), _f32),
                       pltpu.VMEM_SHARED((N, W), _f32)),
        compiler_params=pltpu.CompilerParams(use_tc_tiling_on_sc=False),
    )
    def k(m_hbm, didx_hbm, z_hbm, out_hbm, didx, buf, accum):
        cid = lax.axis_index("c")
        sid = lax.axis_index("s")
        wid = sid * NC + cid
        pltpu.sync_copy(didx_hbm.at[wid], didx)
        pltpu.sync_copy(z_hbm, accum.at[pl.ds(sid * NPC, NPC)])
        plsc.subcore_barrier()

        def body(j, carry):
            base = wid * EPW + j * CH
            pltpu.sync_copy(m_hbm.at[pl.ds(base, CH)], buf)
            pltpu.sync_copy(buf, accum.at[didx.at[j]], add=True)
            return carry

        lax.fori_loop(0, NCHUNK, body, 0)
        plsc.subcore_barrier()
        pltpu.sync_copy(accum.at[pl.ds(sid * NPC, NPC)],
                        out_hbm.at[cid, pl.ds(sid * NPC, NPC)])

    return k


def _sc_scatter(m, didx, z):
    return _build_sc_scatter()(m, didx, z)


@functools.cache
def _build_sc_head_gather():
    @functools.partial(
        pl.kernel,
        out_type=(jax.ShapeDtypeStruct((RP, W), _f32),
                  jax.ShapeDtypeStruct((RP, W), _f32)),
        mesh=_mesh(),
        scratch_types=(pltpu.VMEM((RPW,), jnp.int32),
                       pltpu.VMEM((RPW,), jnp.int32),
                       pltpu.VMEM((RPW, W), _f32),
                       pltpu.VMEM((RPW, W), _f32),
                       pltpu.SemaphoreType.DMA,
                       pltpu.SemaphoreType.DMA),
        compiler_params=pltpu.CompilerParams(use_tc_tiling_on_sc=False),
    )
    def k(a_hbm, b_hbm, es_hbm, ed_hbm, gp_hbm, gq_hbm,
          i1, i2, b1, b2, s1, s2):
        wid = lax.axis_index("s") * NC + lax.axis_index("c")
        pltpu.sync_copy(es_hbm.at[wid], i1)
        pltpu.sync_copy(ed_hbm.at[wid], i2)
        c1 = pltpu.async_copy(a_hbm.at[i1], b1, s1)
        c2 = pltpu.async_copy(b_hbm.at[i2], b2, s2)
        c1.wait()
        c2.wait()
        pltpu.sync_copy(b1, gp_hbm.at[pl.ds(wid * RPW, RPW)])
        pltpu.sync_copy(b2, gq_hbm.at[pl.ds(wid * RPW, RPW)])

    return k


def _sc_head_gather(a, b, esrc, edst):
    return _build_sc_head_gather()(a, b, esrc, edst)


def _edge_mlp(ga, gb, ef, w1, b1, w2, b2, cw1, cb1, cw2, with_ones):
    def body(ga_ref, gb_ref, ef_ref, w1_ref, b1_ref, w2_ref, b2_ref,
             cw1_ref, cb1_ref, cw2_ref, m_ref):
        ha = ga_ref[:, :HID]
        hb = gb_ref[:, :HID]
        xd = ga_ref[:, HID:HID + 3] + gb_ref[:, HID:HID + 3]
        radial = jnp.sum(xd * xd, axis=1, keepdims=True)
        xdn = xd / (jnp.sqrt(radial) + 1e-30)
        f = jnp.concatenate([ha, hb, radial, ef_ref[...]], axis=1)
        pre1 = _bdot(f, w1_ref[...]) + b1_ref[...]
        m = _silu(pre1)
        mh = _silu(_bdot(m, w2_ref[...]) + b2_ref[...])
        cm = _silu(_mdot(mh, cw1_ref[...]) + cb1_ref[...])
        coef = _bdot(cm, cw2_ref[...])
        mx = coef * xdn
        tail = jnp.zeros((EB, 12), _f32)
        if with_ones:
            last = jnp.ones((EB, 1), _f32)
        else:
            last = jnp.zeros((EB, 1), _f32)
        m_ref[...] = jnp.concatenate([mh, mx, tail, last], axis=1)

    full = lambda i: (0, 0)
    return pl.pallas_call(
        body,
        grid=(E // EB,),
        in_specs=[pl.BlockSpec((EB, W), lambda i: (i, 0)),
                  pl.BlockSpec((EB, W), lambda i: (i, 0)),
                  pl.BlockSpec((EB, EDF), lambda i: (i, 0)),
                  pl.BlockSpec((2 * ND + 1 + EDF, HID), full),
                  pl.BlockSpec((1, HID), full),
                  pl.BlockSpec((HID, HID), full),
                  pl.BlockSpec((1, HID), full),
                  pl.BlockSpec((HID, HID), full),
                  pl.BlockSpec((1, HID), full),
                  pl.BlockSpec((HID, 1), full)],
        out_specs=pl.BlockSpec((EB, W), lambda i: (i, 0)),
        out_shape=jax.ShapeDtypeStruct((E, W), _f32),
    )(ga, gb, ef, w1, b1, w2, b2, cw1, cb1, cw2)


def _node_update(h, cpad, s0, s1, deg, nw1, nb1, nw2, nb2, first):
    def body(h_ref, c_ref, s0_ref, s1_ref, d_ref, nw1_ref, nb1_ref,
             nw2_ref, nb2_ref, h_o, c_o, a_o, b_o, d_o):
        s0v = s0_ref[...]
        s1v = s1_ref[...]
        hn = s0v[:, :HID] + s1v[:, :HID]
        xs = s0v[:, HID:HID + 3] + s1v[:, HID:HID + 3]
        if first:
            d = jnp.maximum(s0v[:, W - 1:W] + s1v[:, W - 1:W], 1.0)
        else:
            d = d_ref[...]
        xn = xs / d
        c3 = c_ref[:, :3] + xn
        cnew = jnp.concatenate([c3, jnp.zeros((NB, 13), _f32)], axis=1)
        pre = _bdot(jnp.concatenate([h_ref[...], hn], axis=1), nw1_ref[...]) \
            + nb1_ref[...]
        hnew = _bdot(_silu(pre), nw2_ref[...]) + nb2_ref[...]
        h_o[...] = hnew
        c_o[...] = cnew
        a_o[...] = jnp.concatenate([hnew, cnew], axis=1)
        b_o[...] = jnp.concatenate([hnew, -cnew], axis=1)
        d_o[...] = d

    full = lambda i: (0, 0)
    return pl.pallas_call(
        body,
        grid=(N // NB,),
        in_specs=[pl.BlockSpec((NB, ND), lambda i: (i, 0)),
                  pl.BlockSpec((NB, 16), lambda i: (i, 0)),
                  pl.BlockSpec((NB, W), lambda i: (i, 0)),
                  pl.BlockSpec((NB, W), lambda i: (i, 0)),
                  pl.BlockSpec((NB, 1), lambda i: (i, 0)),
                  pl.BlockSpec((2 * ND, HID), full),
                  pl.BlockSpec((1, HID), full),
                  pl.BlockSpec((HID, HID), full),
                  pl.BlockSpec((1, HID), full)],
        out_specs=[pl.BlockSpec((NB, ND), lambda i: (i, 0)),
                   pl.BlockSpec((NB, 16), lambda i: (i, 0)),
                   pl.BlockSpec((NB, W), lambda i: (i, 0)),
                   pl.BlockSpec((NB, W), lambda i: (i, 0)),
                   pl.BlockSpec((NB, 1), lambda i: (i, 0))],
        out_shape=[jax.ShapeDtypeStruct((N, ND), _f32),
                   jax.ShapeDtypeStruct((N, 16), _f32),
                   jax.ShapeDtypeStruct((N, W), _f32),
                   jax.ShapeDtypeStruct((N, W), _f32),
                   jax.ShapeDtypeStruct((N, 1), _f32)],
    )(h, cpad, s0, s1, deg, nw1, nb1, nw2, nb2)


def _head(gp, gq, hw1, hb1, hw2, hb2):
    def body(gp_ref, gq_ref, w1_ref, b1_ref, w2_ref, b2_ref, o_ref):
        sel = jnp.concatenate([gp_ref[:, :HID], gq_ref[:, :HID]], axis=1)
        s = _silu(_hdot(sel, w1_ref[...]) + b1_ref[...])
        o_ref[...] = _hdot(s, w2_ref[...]) + b2_ref[...]

    full = lambda: (0, 0)
    return pl.pallas_call(
        body,
        in_specs=[pl.BlockSpec((RP, W), full),
                  pl.BlockSpec((RP, W), full),
                  pl.BlockSpec((2 * ND, HID), full),
                  pl.BlockSpec((1, HID), full),
                  pl.BlockSpec((HID, 2), full),
                  pl.BlockSpec((1, 2), full)],
        out_specs=pl.BlockSpec((RP, 2), full),
        out_shape=jax.ShapeDtypeStruct((RP, 2), _f32),
    )(gp, gq, hw1, hb1, hw2, hb2)


def kernel(x, coordinates, edge_features, edge_index, rotatable_idx,
           batch_size, edge_w1, edge_b1, edge_w2, edge_b2, node_w1, node_b1,
           node_w2, node_b2, coord_w1, coord_b1, coord_w2, mlp_w1, mlp_b1,
           mlp_w2, mlp_b2):
    idx2 = edge_index.reshape(2, NW, NCHUNK, CH)
    didx = edge_index[1].reshape(NW, NCHUNK, CH)
    zeros_npc = jnp.zeros((NPC, W), _f32)
    cpad = jnp.pad(coordinates, ((0, 0), (0, 16 - 3)))

    h = x
    deg = jnp.ones((N, 1), _f32)
    A = jnp.concatenate([x, cpad], axis=1)
    B = jnp.concatenate([x, -cpad], axis=1)

    for i in range(NLAYER):
        ga, gb = _sc_gather(A, B, idx2)
        m = _edge_mlp(ga, gb, edge_features, edge_w1[i],
                      edge_b1[i].reshape(1, HID),
                      edge_w2[i], edge_b2[i].reshape(1, HID),
                      coord_w1[i], coord_b1[i].reshape(1, HID),
                      coord_w2[i], with_ones=(i == 0))
        s = _sc_scatter(m, didx, zeros_npc)
        h, cpad, A, B, deg = _node_update(
            h, cpad, s[0], s[1], deg,
            node_w1[i], node_b1[i].reshape(1, HID),
            node_w2[i], node_b2[i].reshape(1, HID), first=(i == 0))

    npad = RP - rotatable_idx.shape[0]
    rix = jnp.concatenate(
        [rotatable_idx, jnp.zeros((npad,), rotatable_idx.dtype)])
    esrc = jnp.take(edge_index[0], rix).reshape(NW, RPW)
    edst = jnp.take(edge_index[1], rix).reshape(NW, RPW)
    gp, gq = _sc_head_gather(A, B, esrc, edst)
    out = _head(gp, gq, mlp_w1, mlp_b1.reshape(1, HID), mlp_w2,
                mlp_b2.reshape(1, 2))
    out = out[:rotatable_idx.shape[0]]
    out = out + 0 * jnp.asarray(batch_size, dtype=out.dtype)
    return out.reshape(rotatable_idx.shape[0] // 2, -1)


# speedup vs baseline: 1.7125x; 1.0658x over previous
import functools

import jax
import jax.numpy as jnp
from jax import lax
from jax.experimental import pallas as pl
from jax.experimental.pallas import tpu as pltpu
from jax.experimental.pallas import tpu_sc as plsc

N = 10000
E = 320000
ND = 128
EDF = 16
HID = 128
W = 144
NLAYER = 7

NC, NS = 2, 16
NW = NC * NS
EPW = E // NW
CH = 80
NCHUNK = EPW // CH
NPC = N // NS

EB = 512
NB = 1000
RP = 1024
RPW = RP // NW

@functools.cache
def _mesh():
    return plsc.VectorSubcoreMesh(core_axis_name="c", subcore_axis_name="s",
                                  num_cores=NC, num_subcores=NS)
_f32 = jnp.float32


def _silu(v):
    return v * jax.nn.sigmoid(v)


def _bdot(a, b):
    return jnp.dot(a.astype(jnp.bfloat16), b.astype(jnp.bfloat16),
                   preferred_element_type=_f32)


def _btrunc(v):
    return v.astype(jnp.bfloat16).astype(_f32)


def _mdot(a, b):
    return jnp.dot(a, b.astype(jnp.bfloat16).astype(_f32),
                   preferred_element_type=_f32,
                   precision=jax.lax.Precision.HIGHEST)


def _hdot(a, b):
    return jnp.dot(a.astype(jnp.bfloat16).astype(_f32), b,
                   preferred_element_type=_f32,
                   precision=jax.lax.Precision.HIGHEST)


@functools.cache
def _build_sc_gather():
    @functools.partial(
        pl.kernel,
        out_type=(jax.ShapeDtypeStruct((E, W), _f32),
                  jax.ShapeDtypeStruct((E, W), _f32)),
        mesh=_mesh(),
        scratch_types=(pltpu.VMEM((NCHUNK, CH), jnp.int32),
                       pltpu.VMEM((NCHUNK, CH), jnp.int32),
                       pltpu.VMEM((CH, W), _f32),
                       pltpu.VMEM((CH, W), _f32),
                       pltpu.VMEM((CH, W), _f32),
                       pltpu.VMEM((CH, W), _f32),
                       pltpu.SemaphoreType.DMA,
                       pltpu.SemaphoreType.DMA,
                       pltpu.SemaphoreType.DMA,
                       pltpu.SemaphoreType.DMA),
        compiler_params=pltpu.CompilerParams(use_tc_tiling_on_sc=False),
    )
    def k(a_hbm, b_hbm, idx_hbm, ga_hbm, gb_hbm, idxs, idxd,
          bufa0, bufb0, bufa1, bufb1, sa0, sb0, sa1, sb1):
        wid = lax.axis_index("s") * NC + lax.axis_index("c")
        pltpu.sync_copy(idx_hbm.at[0, wid], idxs)
        pltpu.sync_copy(idx_hbm.at[1, wid], idxd)

        def issue(j, bufa, bufb, sa, sb):
            pltpu.async_copy(a_hbm.at[idxs.at[j]], bufa, sa)
            pltpu.async_copy(b_hbm.at[idxd.at[j]], bufb, sb)

        def drain_write(j, bufa, bufb, sa, sb):
            pltpu.make_async_copy(a_hbm.at[idxs.at[j]], bufa, sa).wait()
            pltpu.make_async_copy(b_hbm.at[idxd.at[j]], bufb, sb).wait()
            base = wid * EPW + j * CH
            pltpu.sync_copy(bufa, ga_hbm.at[pl.ds(base, CH)])
            pltpu.sync_copy(bufb, gb_hbm.at[pl.ds(base, CH)])

        issue(0, bufa0, bufb0, sa0, sb0)

        def body(g, carry):
            j0 = 2 * g
            issue(j0 + 1, bufa1, bufb1, sa1, sb1)
            drain_write(j0, bufa0, bufb0, sa0, sb0)
            issue(j0 + 2, bufa0, bufb0, sa0, sb0)
            drain_write(j0 + 1, bufa1, bufb1, sa1, sb1)
            return carry

        lax.fori_loop(0, (NCHUNK - 1) // 2, body, 0)
        drain_write(NCHUNK - 1, bufa0, bufb0, sa0, sb0)

    return k


def _sc_gather(a, b, idx2):
    return _build_sc_gather()(a, b, idx2)


@functools.cache
def _build_sc_scatter():
    @functools.partial(
        pl.kernel,
        out_type=jax.ShapeDtypeStruct((NC, N, W), _f32),
        mesh=_mesh(),
        scratch_types=(pltpu.VMEM((NCHUNK, CH), jnp.int32),
                       pltpu.VMEM((CH, W), _f32),
                       pltpu.VMEM((CH, W), _f32),
                       pltpu.VMEM_SHARED((N, W), _f32),
                       pltpu.SemaphoreType.DMA,
                       pltpu.SemaphoreType.DMA),
        compiler_params=pltpu.CompilerParams(use_tc_tiling_on_sc=False),
    )
    def k(m_hbm, didx_hbm, z_hbm, out_hbm, didx, buf0, buf1, accum, sl0, sl1):
        cid = lax.axis_index("c")
        sid = lax.axis_index("s")
        wid = sid * NC + cid
        pltpu.sync_copy(didx_hbm.at[wid], didx)
        pltpu.sync_copy(z_hbm, accum.at[pl.ds(sid * NPC, NPC)])
        plsc.subcore_barrier()

        def issue(j, buf, sem):
            base = wid * EPW + j * CH
            pltpu.async_copy(m_hbm.at[pl.ds(base, CH)], buf, sem)

        def drain_scatter(j, buf, sem):
            base = wid * EPW + j * CH
            pltpu.make_async_copy(m_hbm.at[pl.ds(base, CH)], buf, sem).wait()
            pltpu.sync_copy(buf, accum.at[didx.at[j]], add=True)

        issue(0, buf0, sl0)

        def body(g, carry):
            j0 = 2 * g
            issue(j0 + 1, buf1, sl1)
            drain_scatter(j0, buf0, sl0)
            issue(j0 + 2, buf0, sl0)
            drain_scatter(j0 + 1, buf1, sl1)
            return carry

        lax.fori_loop(0, (NCHUNK - 1) // 2, body, 0)
        drain_scatter(NCHUNK - 1, buf0, sl0)
        plsc.subcore_barrier()
        pltpu.sync_copy(accum.at[pl.ds(sid * NPC, NPC)],
                        out_hbm.at[cid, pl.ds(sid * NPC, NPC)])

    return k


def _sc_scatter(m, didx, z):
    return _build_sc_scatter()(m, didx, z)


@functools.cache
def _build_sc_head_gather():
    @functools.partial(
        pl.kernel,
        out_type=(jax.ShapeDtypeStruct((RP, W), _f32),
                  jax.ShapeDtypeStruct((RP, W), _f32)),
        mesh=_mesh(),
        scratch_types=(pltpu.VMEM((RPW,), jnp.int32),
                       pltpu.VMEM((RPW,), jnp.int32),
                       pltpu.VMEM((RPW, W), _f32),
                       pltpu.VMEM((RPW, W), _f32),
                       pltpu.SemaphoreType.DMA,
                       pltpu.SemaphoreType.DMA),
        compiler_params=pltpu.CompilerParams(use_tc_tiling_on_sc=False),
    )
    def k(a_hbm, b_hbm, es_hbm, ed_hbm, gp_hbm, gq_hbm,
          i1, i2, b1, b2, s1, s2):
        wid = lax.axis_index("s") * NC + lax.axis_index("c")
        pltpu.sync_copy(es_hbm.at[wid], i1)
        pltpu.sync_copy(ed_hbm.at[wid], i2)
        c1 = pltpu.async_copy(a_hbm.at[i1], b1, s1)
        c2 = pltpu.async_copy(b_hbm.at[i2], b2, s2)
        c1.wait()
        c2.wait()
        pltpu.sync_copy(b1, gp_hbm.at[pl.ds(wid * RPW, RPW)])
        pltpu.sync_copy(b2, gq_hbm.at[pl.ds(wid * RPW, RPW)])

    return k


def _sc_head_gather(a, b, esrc, edst):
    return _build_sc_head_gather()(a, b, esrc, edst)


def _edge_mlp(ga, gb, ef, w1, b1, w2, b2, cw1, cb1, cw2, with_ones):
    def body(ga_ref, gb_ref, ef_ref, w1_ref, b1_ref, w2_ref, b2_ref,
             cw1_ref, cb1_ref, cw2_ref, m_ref):
        ha = ga_ref[:, :HID]
        hb = gb_ref[:, :HID]
        xd = ga_ref[:, HID:HID + 3] + gb_ref[:, HID:HID + 3]
        radial = jnp.sum(xd * xd, axis=1, keepdims=True)
        xdn = xd / (jnp.sqrt(radial) + 1e-30)
        f = jnp.concatenate([ha, hb, radial, ef_ref[...]], axis=1)
        pre1 = _bdot(f, w1_ref[...]) + b1_ref[...]
        m = _silu(pre1)
        mh = _silu(_bdot(m, w2_ref[...]) + b2_ref[...])
        cm = _silu(_mdot(mh, cw1_ref[...]) + cb1_ref[...])
        coef = _bdot(cm, cw2_ref[...])
        mx = coef * xdn
        tail = jnp.zeros((EB, 12), _f32)
        if with_ones:
            last = jnp.ones((EB, 1), _f32)
        else:
            last = jnp.zeros((EB, 1), _f32)
        m_ref[...] = jnp.concatenate([mh, mx, tail, last], axis=1)

    full = lambda i: (0, 0)
    return pl.pallas_call(
        body,
        grid=(E // EB,),
        in_specs=[pl.BlockSpec((EB, W), lambda i: (i, 0)),
                  pl.BlockSpec((EB, W), lambda i: (i, 0)),
                  pl.BlockSpec((EB, EDF), lambda i: (i, 0)),
                  pl.BlockSpec((2 * ND + 1 + EDF, HID), full),
                  pl.BlockSpec((1, HID), full),
                  pl.BlockSpec((HID, HID), full),
                  pl.BlockSpec((1, HID), full),
                  pl.BlockSpec((HID, HID), full),
                  pl.BlockSpec((1, HID), full),
                  pl.BlockSpec((HID, 1), full)],
        out_specs=pl.BlockSpec((EB, W), lambda i: (i, 0)),
        out_shape=jax.ShapeDtypeStruct((E, W), _f32),
    )(ga, gb, ef, w1, b1, w2, b2, cw1, cb1, cw2)


def _node_update(h, cpad, s0, s1, deg, nw1, nb1, nw2, nb2, first):
    def body(h_ref, c_ref, s0_ref, s1_ref, d_ref, nw1_ref, nb1_ref,
             nw2_ref, nb2_ref, h_o, c_o, a_o, b_o, d_o):
        s0v = s0_ref[...]
        s1v = s1_ref[...]
        hn = s0v[:, :HID] + s1v[:, :HID]
        xs = s0v[:, HID:HID + 3] + s1v[:, HID:HID + 3]
        if first:
            d = jnp.maximum(s0v[:, W - 1:W] + s1v[:, W - 1:W], 1.0)
        else:
            d = d_ref[...]
        xn = xs / d
        c3 = c_ref[:, :3] + xn
        cnew = jnp.concatenate([c3, jnp.zeros((NB, 13), _f32)], axis=1)
        pre = _bdot(jnp.concatenate([h_ref[...], hn], axis=1), nw1_ref[...]) \
            + nb1_ref[...]
        hnew = _bdot(_silu(pre), nw2_ref[...]) + nb2_ref[...]
        h_o[...] = hnew
        c_o[...] = cnew
        a_o[...] = jnp.concatenate([hnew, cnew], axis=1)
        b_o[...] = jnp.concatenate([hnew, -cnew], axis=1)
        d_o[...] = d

    full = lambda i: (0, 0)
    return pl.pallas_call(
        body,
        grid=(N // NB,),
        in_specs=[pl.BlockSpec((NB, ND), lambda i: (i, 0)),
                  pl.BlockSpec((NB, 16), lambda i: (i, 0)),
                  pl.BlockSpec((NB, W), lambda i: (i, 0)),
                  pl.BlockSpec((NB, W), lambda i: (i, 0)),
                  pl.BlockSpec((NB, 1), lambda i: (i, 0)),
                  pl.BlockSpec((2 * ND, HID), full),
                  pl.BlockSpec((1, HID), full),
                  pl.BlockSpec((HID, HID), full),
                  pl.BlockSpec((1, HID), full)],
        out_specs=[pl.BlockSpec((NB, ND), lambda i: (i, 0)),
                   pl.BlockSpec((NB, 16), lambda i: (i, 0)),
                   pl.BlockSpec((NB, W), lambda i: (i, 0)),
                   pl.BlockSpec((NB, W), lambda i: (i, 0)),
                   pl.BlockSpec((NB, 1), lambda i: (i, 0))],
        out_shape=[jax.ShapeDtypeStruct((N, ND), _f32),
                   jax.ShapeDtypeStruct((N, 16), _f32),
                   jax.ShapeDtypeStruct((N, W), _f32),
                   jax.ShapeDtypeStruct((N, W), _f32),
                   jax.ShapeDtypeStruct((N, 1), _f32)],
    )(h, cpad, s0, s1, deg, nw1, nb1, nw2, nb2)


def _head(gp, gq, hw1, hb1, hw2, hb2):
    def body(gp_ref, gq_ref, w1_ref, b1_ref, w2_ref, b2_ref, o_ref):
        sel = jnp.concatenate([gp_ref[:, :HID], gq_ref[:, :HID]], axis=1)
        s = _silu(_hdot(sel, w1_ref[...]) + b1_ref[...])
        o_ref[...] = _hdot(s, w2_ref[...]) + b2_ref[...]

    full = lambda: (0, 0)
    return pl.pallas_call(
        body,
        in_specs=[pl.BlockSpec((RP, W), full),
                  pl.BlockSpec((RP, W), full),
                  pl.BlockSpec((2 * ND, HID), full),
                  pl.BlockSpec((1, HID), full),
                  pl.BlockSpec((HID, 2), full),
                  pl.BlockSpec((1, 2), full)],
        out_specs=pl.BlockSpec((RP, 2), full),
        out_shape=jax.ShapeDtypeStruct((RP, 2), _f32),
    )(gp, gq, hw1, hb1, hw2, hb2)


def kernel(x, coordinates, edge_features, edge_index, rotatable_idx,
           batch_size, edge_w1, edge_b1, edge_w2, edge_b2, node_w1, node_b1,
           node_w2, node_b2, coord_w1, coord_b1, coord_w2, mlp_w1, mlp_b1,
           mlp_w2, mlp_b2):
    idx2 = edge_index.reshape(2, NW, NCHUNK, CH)
    didx = edge_index[1].reshape(NW, NCHUNK, CH)
    zeros_npc = jnp.zeros((NPC, W), _f32)
    cpad = jnp.pad(coordinates, ((0, 0), (0, 16 - 3)))

    h = x
    deg = jnp.ones((N, 1), _f32)
    A = jnp.concatenate([x, cpad], axis=1)
    B = jnp.concatenate([x, -cpad], axis=1)

    for i in range(NLAYER):
        ga, gb = _sc_gather(A, B, idx2)
        m = _edge_mlp(ga, gb, edge_features, edge_w1[i],
                      edge_b1[i].reshape(1, HID),
                      edge_w2[i], edge_b2[i].reshape(1, HID),
                      coord_w1[i], coord_b1[i].reshape(1, HID),
                      coord_w2[i], with_ones=(i == 0))
        s = _sc_scatter(m, didx, zeros_npc)
        h, cpad, A, B, deg = _node_update(
            h, cpad, s[0], s[1], deg,
            node_w1[i], node_b1[i].reshape(1, HID),
            node_w2[i], node_b2[i].reshape(1, HID), first=(i == 0))

    npad = RP - rotatable_idx.shape[0]
    rix = jnp.concatenate(
        [rotatable_idx, jnp.zeros((npad,), rotatable_idx.dtype)])
    esrc = jnp.take(edge_index[0], rix).reshape(NW, RPW)
    edst = jnp.take(edge_index[1], rix).reshape(NW, RPW)
    gp, gq = _sc_head_gather(A, B, esrc, edst)
    out = _head(gp, gq, mlp_w1, mlp_b1.reshape(1, HID), mlp_w2,
                mlp_b2.reshape(1, 2))
    out = out[:rotatable_idx.shape[0]]
    out = out + 0 * jnp.asarray(batch_size, dtype=out.dtype)
    return out.reshape(rotatable_idx.shape[0] // 2, -1)


# speedup vs baseline: 1.7594x; 1.0274x over previous
import functools

import jax
import jax.numpy as jnp
from jax import lax
from jax.experimental import pallas as pl
from jax.experimental.pallas import tpu as pltpu
from jax.experimental.pallas import tpu_sc as plsc

N = 10000
E = 320000
ND = 128
EDF = 16
HID = 128
W = 144
NLAYER = 7

NC, NS = 2, 16
NW = NC * NS
EPW = E // NW
CH = 80
NCHUNK = EPW // CH
NPC = N // NS

EB = 512
NB = 1000
RP = 1024
RPW = RP // NW

@functools.cache
def _mesh():
    return plsc.VectorSubcoreMesh(core_axis_name="c", subcore_axis_name="s",
                                  num_cores=NC, num_subcores=NS)
_f32 = jnp.float32


def _silu(v):
    return v * jax.nn.sigmoid(v)


def _bdot(a, b):
    return jnp.dot(a.astype(jnp.bfloat16), b.astype(jnp.bfloat16),
                   preferred_element_type=_f32)


def _btrunc(v):
    return v.astype(jnp.bfloat16).astype(_f32)


def _mdot(a, b):
    hi = _btrunc(a)
    lo = a - hi
    return _bdot(hi, b) + _bdot(lo, b)


def _hdot(a, b):
    hi = _btrunc(b)
    lo = b - hi
    return _bdot(a, hi) + _bdot(a, lo)


@functools.cache
def _build_sc_gather():
    @functools.partial(
        pl.kernel,
        out_type=(jax.ShapeDtypeStruct((E, W), _f32),
                  jax.ShapeDtypeStruct((E, W), _f32)),
        mesh=_mesh(),
        scratch_types=(pltpu.VMEM((NCHUNK, CH), jnp.int32),
                       pltpu.VMEM((NCHUNK, CH), jnp.int32),
                       pltpu.VMEM((CH, W), _f32),
                       pltpu.VMEM((CH, W), _f32),
                       pltpu.VMEM((CH, W), _f32),
                       pltpu.VMEM((CH, W), _f32),
                       pltpu.SemaphoreType.DMA,
                       pltpu.SemaphoreType.DMA,
                       pltpu.SemaphoreType.DMA,
                       pltpu.SemaphoreType.DMA),
        compiler_params=pltpu.CompilerParams(use_tc_tiling_on_sc=False),
    )
    def k(a_hbm, b_hbm, idx_hbm, ga_hbm, gb_hbm, idxs, idxd,
          bufa0, bufb0, bufa1, bufb1, sa0, sb0, sa1, sb1):
        wid = lax.axis_index("s") * NC + lax.axis_index("c")
        pltpu.sync_copy(idx_hbm.at[0, wid], idxs)
        pltpu.sync_copy(idx_hbm.at[1, wid], idxd)

        def issue(j, bufa, bufb, sa, sb):
            pltpu.async_copy(a_hbm.at[idxs.at[j]], bufa, sa)
            pltpu.async_copy(b_hbm.at[idxd.at[j]], bufb, sb)

        def drain_write(j, bufa, bufb, sa, sb):
            pltpu.make_async_copy(a_hbm.at[idxs.at[j]], bufa, sa).wait()
            pltpu.make_async_copy(b_hbm.at[idxd.at[j]], bufb, sb).wait()
            base = wid * EPW + j * CH
            pltpu.sync_copy(bufa, ga_hbm.at[pl.ds(base, CH)])
            pltpu.sync_copy(bufb, gb_hbm.at[pl.ds(base, CH)])

        issue(0, bufa0, bufb0, sa0, sb0)

        def body(g, carry):
            j0 = 2 * g
            issue(j0 + 1, bufa1, bufb1, sa1, sb1)
            drain_write(j0, bufa0, bufb0, sa0, sb0)
            issue(j0 + 2, bufa0, bufb0, sa0, sb0)
            drain_write(j0 + 1, bufa1, bufb1, sa1, sb1)
            return carry

        lax.fori_loop(0, (NCHUNK - 1) // 2, body, 0)
        drain_write(NCHUNK - 1, bufa0, bufb0, sa0, sb0)

    return k


def _sc_gather(a, b, idx2):
    return _build_sc_gather()(a, b, idx2)


@functools.cache
def _build_sc_scatter():
    @functools.partial(
        pl.kernel,
        out_type=jax.ShapeDtypeStruct((NC, N, W), _f32),
        mesh=_mesh(),
        scratch_types=(pltpu.VMEM((NCHUNK, CH), jnp.int32),
                       pltpu.VMEM((CH, W), _f32),
                       pltpu.VMEM((CH, W), _f32),
                       pltpu.VMEM_SHARED((N, W), _f32),
                       pltpu.SemaphoreType.DMA,
                       pltpu.SemaphoreType.DMA),
        compiler_params=pltpu.CompilerParams(use_tc_tiling_on_sc=False),
    )
    def k(m_hbm, didx_hbm, z_hbm, out_hbm, didx, buf0, buf1, accum, sl0, sl1):
        cid = lax.axis_index("c")
        sid = lax.axis_index("s")
        wid = sid * NC + cid
        pltpu.sync_copy(didx_hbm.at[wid], didx)
        pltpu.sync_copy(z_hbm, accum.at[pl.ds(sid * NPC, NPC)])
        plsc.subcore_barrier()

        def issue(j, buf, sem):
            base = wid * EPW + j * CH
            pltpu.async_copy(m_hbm.at[pl.ds(base, CH)], buf, sem)

        def drain_scatter(j, buf, sem):
            base = wid * EPW + j * CH
            pltpu.make_async_copy(m_hbm.at[pl.ds(base, CH)], buf, sem).wait()
            pltpu.sync_copy(buf, accum.at[didx.at[j]], add=True)

        issue(0, buf0, sl0)

        def body(g, carry):
            j0 = 2 * g
            issue(j0 + 1, buf1, sl1)
            drain_scatter(j0, buf0, sl0)
            issue(j0 + 2, buf0, sl0)
            drain_scatter(j0 + 1, buf1, sl1)
            return carry

        lax.fori_loop(0, (NCHUNK - 1) // 2, body, 0)
        drain_scatter(NCHUNK - 1, buf0, sl0)
        plsc.subcore_barrier()
        pltpu.sync_copy(accum.at[pl.ds(sid * NPC, NPC)],
                        out_hbm.at[cid, pl.ds(sid * NPC, NPC)])

    return k


def _sc_scatter(m, didx, z):
    return _build_sc_scatter()(m, didx, z)


@functools.cache
def _build_sc_head_gather():
    @functools.partial(
        pl.kernel,
        out_type=(jax.ShapeDtypeStruct((RP, W), _f32),
                  jax.ShapeDtypeStruct((RP, W), _f32)),
        mesh=_mesh(),
        scratch_types=(pltpu.VMEM((RPW,), jnp.int32),
                       pltpu.VMEM((RPW,), jnp.int32),
                       pltpu.VMEM((RPW, W), _f32),
                       pltpu.VMEM((RPW, W), _f32),
                       pltpu.SemaphoreType.DMA,
                       pltpu.SemaphoreType.DMA),
        compiler_params=pltpu.CompilerParams(use_tc_tiling_on_sc=False),
    )
    def k(a_hbm, b_hbm, es_hbm, ed_hbm, gp_hbm, gq_hbm,
          i1, i2, b1, b2, s1, s2):
        wid = lax.axis_index("s") * NC + lax.axis_index("c")
        pltpu.sync_copy(es_hbm.at[wid], i1)
        pltpu.sync_copy(ed_hbm.at[wid], i2)
        c1 = pltpu.async_copy(a_hbm.at[i1], b1, s1)
        c2 = pltpu.async_copy(b_hbm.at[i2], b2, s2)
        c1.wait()
        c2.wait()
        pltpu.sync_copy(b1, gp_hbm.at[pl.ds(wid * RPW, RPW)])
        pltpu.sync_copy(b2, gq_hbm.at[pl.ds(wid * RPW, RPW)])

    return k


def _sc_head_gather(a, b, esrc, edst):
    return _build_sc_head_gather()(a, b, esrc, edst)


def _edge_mlp(ga, gb, ef, w1, b1, w2, b2, cw1, cb1, cw2, with_ones):
    def body(ga_ref, gb_ref, ef_ref, w1_ref, b1_ref, w2_ref, b2_ref,
             cw1_ref, cb1_ref, cw2_ref, m_ref):
        ha = ga_ref[:, :HID]
        hb = gb_ref[:, :HID]
        xd = ga_ref[:, HID:HID + 3] + gb_ref[:, HID:HID + 3]
        radial = jnp.sum(xd * xd, axis=1, keepdims=True)
        xdn = xd / (jnp.sqrt(radial) + 1e-30)
        f = jnp.concatenate([ha, hb, radial, ef_ref[...]], axis=1)
        pre1 = _bdot(f, w1_ref[...]) + b1_ref[...]
        m = _silu(pre1)
        mh = _silu(_bdot(m, w2_ref[...]) + b2_ref[...])
        cm = _silu(_mdot(mh, cw1_ref[...]) + cb1_ref[...])
        coef = _bdot(cm, cw2_ref[...])
        mx = coef * xdn
        tail = jnp.zeros((EB, 12), _f32)
        if with_ones:
            last = jnp.ones((EB, 1), _f32)
        else:
            last = jnp.zeros((EB, 1), _f32)
        m_ref[...] = jnp.concatenate([mh, mx, tail, last], axis=1)

    full = lambda i: (0, 0)
    return pl.pallas_call(
        body,
        grid=(E // EB,),
        in_specs=[pl.BlockSpec((EB, W), lambda i: (i, 0)),
                  pl.BlockSpec((EB, W), lambda i: (i, 0)),
                  pl.BlockSpec((EB, EDF), lambda i: (i, 0)),
                  pl.BlockSpec((2 * ND + 1 + EDF, HID), full),
                  pl.BlockSpec((1, HID), full),
                  pl.BlockSpec((HID, HID), full),
                  pl.BlockSpec((1, HID), full),
                  pl.BlockSpec((HID, HID), full),
                  pl.BlockSpec((1, HID), full),
                  pl.BlockSpec((HID, 1), full)],
        out_specs=pl.BlockSpec((EB, W), lambda i: (i, 0)),
        out_shape=jax.ShapeDtypeStruct((E, W), _f32),
    )(ga, gb, ef, w1, b1, w2, b2, cw1, cb1, cw2)


def _node_update(h, cpad, s0, s1, deg, nw1, nb1, nw2, nb2, first):
    def body(h_ref, c_ref, s0_ref, s1_ref, d_ref, nw1_ref, nb1_ref,
             nw2_ref, nb2_ref, h_o, c_o, a_o, b_o, d_o):
        s0v = s0_ref[...]
        s1v = s1_ref[...]
        hn = s0v[:, :HID] + s1v[:, :HID]
        xs = s0v[:, HID:HID + 3] + s1v[:, HID:HID + 3]
        if first:
            d = jnp.maximum(s0v[:, W - 1:W] + s1v[:, W - 1:W], 1.0)
        else:
            d = d_ref[...]
        xn = xs / d
        c3 = c_ref[:, :3] + xn
        cnew = jnp.concatenate([c3, jnp.zeros((NB, 13), _f32)], axis=1)
        pre = _bdot(jnp.concatenate([h_ref[...], hn], axis=1), nw1_ref[...]) \
            + nb1_ref[...]
        hnew = _bdot(_silu(pre), nw2_ref[...]) + nb2_ref[...]
        h_o[...] = hnew
        c_o[...] = cnew
        a_o[...] = jnp.concatenate([hnew, cnew], axis=1)
        b_o[...] = jnp.concatenate([hnew, -cnew], axis=1)
        d_o[...] = d

    full = lambda i: (0, 0)
    return pl.pallas_call(
        body,
        grid=(N // NB,),
        in_specs=[pl.BlockSpec((NB, ND), lambda i: (i, 0)),
                  pl.BlockSpec((NB, 16), lambda i: (i, 0)),
                  pl.BlockSpec((NB, W), lambda i: (i, 0)),
                  pl.BlockSpec((NB, W), lambda i: (i, 0)),
                  pl.BlockSpec((NB, 1), lambda i: (i, 0)),
                  pl.BlockSpec((2 * ND, HID), full),
                  pl.BlockSpec((1, HID), full),
                  pl.BlockSpec((HID, HID), full),
                  pl.BlockSpec((1, HID), full)],
        out_specs=[pl.BlockSpec((NB, ND), lambda i: (i, 0)),
                   pl.BlockSpec((NB, 16), lambda i: (i, 0)),
                   pl.BlockSpec((NB, W), lambda i: (i, 0)),
                   pl.BlockSpec((NB, W), lambda i: (i, 0)),
                   pl.BlockSpec((NB, 1), lambda i: (i, 0))],
        out_shape=[jax.ShapeDtypeStruct((N, ND), _f32),
                   jax.ShapeDtypeStruct((N, 16), _f32),
                   jax.ShapeDtypeStruct((N, W), _f32),
                   jax.ShapeDtypeStruct((N, W), _f32),
                   jax.ShapeDtypeStruct((N, 1), _f32)],
    )(h, cpad, s0, s1, deg, nw1, nb1, nw2, nb2)


def _head(gp, gq, hw1, hb1, hw2, hb2):
    def body(gp_ref, gq_ref, w1_ref, b1_ref, w2_ref, b2_ref, o_ref):
        sel = jnp.concatenate([gp_ref[:, :HID], gq_ref[:, :HID]], axis=1)
        s = _silu(_hdot(sel, w1_ref[...]) + b1_ref[...])
        o_ref[...] = _hdot(s, w2_ref[...]) + b2_ref[...]

    full = lambda: (0, 0)
    return pl.pallas_call(
        body,
        in_specs=[pl.BlockSpec((RP, W), full),
                  pl.BlockSpec((RP, W), full),
                  pl.BlockSpec((2 * ND, HID), full),
                  pl.BlockSpec((1, HID), full),
                  pl.BlockSpec((HID, 2), full),
                  pl.BlockSpec((1, 2), full)],
        out_specs=pl.BlockSpec((RP, 2), full),
        out_shape=jax.ShapeDtypeStruct((RP, 2), _f32),
    )(gp, gq, hw1, hb1, hw2, hb2)


def kernel(x, coordinates, edge_features, edge_index, rotatable_idx,
           batch_size, edge_w1, edge_b1, edge_w2, edge_b2, node_w1, node_b1,
           node_w2, node_b2, coord_w1, coord_b1, coord_w2, mlp_w1, mlp_b1,
           mlp_w2, mlp_b2):
    idx2 = edge_index.reshape(2, NW, NCHUNK, CH)
    didx = edge_index[1].reshape(NW, NCHUNK, CH)
    zeros_npc = jnp.zeros((NPC, W), _f32)
    cpad = jnp.pad(coordinates, ((0, 0), (0, 16 - 3)))

    h = x
    deg = jnp.ones((N, 1), _f32)
    A = jnp.concatenate([x, cpad], axis=1)
    B = jnp.concatenate([x, -cpad], axis=1)

    for i in range(NLAYER):
        ga, gb = _sc_gather(A, B, idx2)
        m = _edge_mlp(ga, gb, edge_features, edge_w1[i],
                      edge_b1[i].reshape(1, HID),
                      edge_w2[i], edge_b2[i].reshape(1, HID),
                      coord_w1[i], coord_b1[i].reshape(1, HID),
                      coord_w2[i], with_ones=(i == 0))
        s = _sc_scatter(m, didx, zeros_npc)
        h, cpad, A, B, deg = _node_update(
            h, cpad, s[0], s[1], deg,
            node_w1[i], node_b1[i].reshape(1, HID),
            node_w2[i], node_b2[i].reshape(1, HID), first=(i == 0))

    npad = RP - rotatable_idx.shape[0]
    rix = jnp.concatenate(
        [rotatable_idx, jnp.zeros((npad,), rotatable_idx.dtype)])
    esrc = jnp.take(edge_index[0], rix).reshape(NW, RPW)
    edst = jnp.take(edge_index[1], rix).reshape(NW, RPW)
    gp, gq = _sc_head_gather(A, B, esrc, edst)
    out = _head(gp, gq, mlp_w1, mlp_b1.reshape(1, HID), mlp_w2,
                mlp_b2.reshape(1, 2))
    out = out[:rotatable_idx.shape[0]]
    out = out + 0 * jnp.asarray(batch_size, dtype=out.dtype)
    return out.reshape(rotatable_idx.shape[0] // 2, -1)
